# baseline (speedup 1.0000x reference)
import functools

import jax
import jax.numpy as jnp
from jax import lax
from jax.experimental import pallas as pl
from jax.experimental.pallas import tpu as pltpu, tpu_sc as plsc

N = 10000
D = 128
DOUT = 40
E = 320000

NC = 2
NS = 16
NW = NC * NS
CH = 128
NPAD = 10240
RPT = NPAD // NS
TRASH = 10016
PAIRS = 40
CHUNKS = 2 * PAIRS
TOT_E = NW * CH * CHUNKS
PAD_E = TOT_E - E
BR = 1024

_mesh = plsc.VectorSubcoreMesh(
    core_axis_name="c", subcore_axis_name="s", num_cores=NC, num_subcores=NS)


@functools.partial(
    pl.kernel,
    out_type=jax.ShapeDtypeStruct((NC * NPAD,), jnp.float32),
    mesh=_mesh,
    scratch_types=[
        pltpu.VMEM((2, CH), jnp.int32),
        pltpu.VMEM((2, CH), jnp.int32),
        pltpu.VMEM((CH,), jnp.float32),
        pltpu.VMEM((RPT,), jnp.float32),
        pltpu.VMEM_SHARED((NPAD,), jnp.float32),
        pltpu.SemaphoreType.DMA,
        pltpu.SemaphoreType.DMA,
        pltpu.SemaphoreType.DMA,
        pltpu.SemaphoreType.DMA,
    ],
)
def _deg_kernel(ei_hbm, out_hbm, iba, ibb, ones_v, zero_v, deg_s,
                isa, isb, ssa, ssb):
    cid = lax.axis_index("c")
    sid = lax.axis_index("s")
    wid = cid * NS + sid
    rbase = sid * RPT
    for i in range(CH // 16):
        ones_v[pl.ds(i * 16, 16)] = jnp.full((16,), 1.0, jnp.float32)
    for i in range(RPT // 16):
        zero_v[pl.ds(i * 16, 16)] = jnp.zeros((16,), jnp.float32)
    pltpu.sync_copy(zero_v, deg_s.at[pl.ds(rbase, RPT)])
    plsc.subcore_barrier()

    def body(i, carry):
        a = (wid * CHUNKS + 2 * i) * 2
        @pl.when(i > 0)
        def _():
            pltpu.make_async_copy(ones_v, deg_s.at[iba.at[1]], ssa).wait()
        pltpu.async_copy(ei_hbm.at[pl.ds(a, 2)], iba, isa)
        pltpu.make_async_copy(ei_hbm.at[pl.ds(a, 2)], iba, isa).wait()
        pltpu.async_copy(ones_v, deg_s.at[iba.at[1]], ssa, add=True)
        @pl.when(i > 0)
        def _():
            pltpu.make_async_copy(ones_v, deg_s.at[ibb.at[1]], ssb).wait()
        pltpu.async_copy(ei_hbm.at[pl.ds(a + 2, 2)], ibb, isb)
        pltpu.make_async_copy(ei_hbm.at[pl.ds(a + 2, 2)], ibb, isb).wait()
        pltpu.async_copy(ones_v, deg_s.at[ibb.at[1]], ssb, add=True)
        return carry

    lax.fori_loop(0, PAIRS, body, 0)
    pltpu.make_async_copy(ones_v, deg_s.at[iba.at[1]], ssa).wait()
    pltpu.make_async_copy(ones_v, deg_s.at[ibb.at[1]], ssb).wait()
    plsc.subcore_barrier()
    pltpu.sync_copy(deg_s.at[pl.ds(rbase, RPT)],
                    out_hbm.at[pl.ds(cid * NPAD + rbase, RPT)])


@functools.partial(
    pl.kernel,
    out_type=jax.ShapeDtypeStruct((NC * NPAD, D), jnp.float32),
    mesh=_mesh,
    scratch_types=[
        pltpu.VMEM((2, CH), jnp.int32),
        pltpu.VMEM((2, CH), jnp.int32),
        pltpu.VMEM((CH, D), jnp.float32),
        pltpu.VMEM((CH, D), jnp.float32),
        pltpu.VMEM_SHARED((NPAD, D), jnp.float32),
        pltpu.SemaphoreType.DMA,
        pltpu.SemaphoreType.DMA,
        pltpu.SemaphoreType.DMA,
        pltpu.SemaphoreType.DMA,
        pltpu.SemaphoreType.DMA,
        pltpu.SemaphoreType.DMA,
    ],
)
def _scatter_kernel(g_hbm, ei_hbm, zsrc_hbm, out_hbm,
                    iba, ibb, rowsa, rowsb, acc_s,
                    isa, isb, gsa, gsb, ssa, ssb):
    cid = lax.axis_index("c")
    sid = lax.axis_index("s")
    wid = cid * NS + sid
    rbase = sid * RPT

    @pl.when(cid == 0)
    def _():
        pltpu.sync_copy(g_hbm.at[pl.ds(rbase, RPT)], acc_s.at[pl.ds(rbase, RPT)])

    @pl.when(cid != 0)
    def _():
        pltpu.sync_copy(zsrc_hbm, acc_s.at[pl.ds(rbase, RPT)])

    plsc.subcore_barrier()

    def body(i, carry):
        a = (wid * CHUNKS + 2 * i) * 2
        @pl.when(i > 0)
        def _():
            pltpu.make_async_copy(rowsa, acc_s.at[iba.at[1]], ssa).wait()
        pltpu.async_copy(ei_hbm.at[pl.ds(a, 2)], iba, isa)
        pltpu.make_async_copy(ei_hbm.at[pl.ds(a, 2)], iba, isa).wait()
        pltpu.async_copy(g_hbm.at[iba.at[0]], rowsa, gsa)
        @pl.when(i > 0)
        def _():
            pltpu.make_async_copy(rowsb, acc_s.at[ibb.at[1]], ssb).wait()
        pltpu.async_copy(ei_hbm.at[pl.ds(a + 2, 2)], ibb, isb)
        pltpu.make_async_copy(ei_hbm.at[pl.ds(a + 2, 2)], ibb, isb).wait()
        pltpu.async_copy(g_hbm.at[ibb.at[0]], rowsb, gsb)
        pltpu.make_async_copy(g_hbm.at[iba.at[0]], rowsa, gsa).wait()
        pltpu.async_copy(rowsa, acc_s.at[iba.at[1]], ssa, add=True)
        pltpu.make_async_copy(g_hbm.at[ibb.at[0]], rowsb, gsb).wait()
        pltpu.async_copy(rowsb, acc_s.at[ibb.at[1]], ssb, add=True)
        return carry

    lax.fori_loop(0, PAIRS, body, 0)
    pltpu.make_async_copy(rowsa, acc_s.at[iba.at[1]], ssa).wait()
    pltpu.make_async_copy(rowsb, acc_s.at[ibb.at[1]], ssb).wait()
    plsc.subcore_barrier()
    pltpu.sync_copy(acc_s.at[pl.ds(rbase, RPT)],
                    out_hbm.at[pl.ds(cid * NPAD + rbase, RPT)])


def _prep_body(x_ref, d0_ref, d1_ref, g_ref, dinv_ref, dginv_ref):
    deg = d0_ref[...] + d1_ref[...] + 1.0
    dv = lax.rsqrt(deg)
    g_ref[...] = x_ref[...] * dv
    dinv_ref[...] = dv
    dginv_ref[...] = dv * dv


_prep = pl.pallas_call(
    _prep_body,
    grid=(NPAD // BR,),
    in_specs=[
        pl.BlockSpec((BR, D), lambda i: (i, 0)),
        pl.BlockSpec((BR, 1), lambda i: (i, 0)),
        pl.BlockSpec((BR, 1), lambda i: (i, 0)),
    ],
    out_specs=[
        pl.BlockSpec((BR, D), lambda i: (i, 0)),
        pl.BlockSpec((BR, 1), lambda i: (i, 0)),
        pl.BlockSpec((BR, 1), lambda i: (i, 0)),
    ],
    out_shape=[
        jax.ShapeDtypeStruct((NPAD, D), jnp.float32),
        jax.ShapeDtypeStruct((NPAD, 1), jnp.float32),
        jax.ShapeDtypeStruct((NPAD, 1), jnp.float32),
    ],
)


def _combine_body(s0_ref, s1_ref, dginv_ref, g_ref):
    g_ref[...] = (s0_ref[...] + s1_ref[...]) * dginv_ref[...]


_combine = pl.pallas_call(
    _combine_body,
    grid=(NPAD // BR,),
    in_specs=[
        pl.BlockSpec((BR, D), lambda i: (i, 0)),
        pl.BlockSpec((BR, D), lambda i: (i, 0)),
        pl.BlockSpec((BR, 1), lambda i: (i, 0)),
    ],
    out_specs=pl.BlockSpec((BR, D), lambda i: (i, 0)),
    out_shape=jax.ShapeDtypeStruct((NPAD, D), jnp.float32),
)


def _final_body(s0_ref, s1_ref, dinv_ref, wt_ref, b_ref, out_ref, emb_ref):
    h2 = (s0_ref[...] + s1_ref[...]) * dinv_ref[...]
    emb = jnp.dot(h2, wt_ref[...], preferred_element_type=jnp.float32) + b_ref[...]
    col = lax.broadcasted_iota(jnp.int32, emb.shape, 1)
    logits = jnp.where(col < DOUT, emb, -1e30)
    m = jnp.max(logits, axis=1, keepdims=True)
    lse = jnp.log(jnp.sum(jnp.exp(logits - m), axis=1, keepdims=True)) + m
    emb_ref[...] = emb
    out_ref[...] = emb - lse


_final = pl.pallas_call(
    _final_body,
    grid=(NPAD // BR,),
    in_specs=[
        pl.BlockSpec((BR, D), lambda i: (i, 0)),
        pl.BlockSpec((BR, D), lambda i: (i, 0)),
        pl.BlockSpec((BR, 1), lambda i: (i, 0)),
        pl.BlockSpec((D, D), lambda i: (0, 0)),
        pl.BlockSpec((1, D), lambda i: (0, 0)),
    ],
    out_specs=[
        pl.BlockSpec((BR, D), lambda i: (i, 0)),
        pl.BlockSpec((BR, D), lambda i: (i, 0)),
    ],
    out_shape=[
        jax.ShapeDtypeStruct((NPAD, D), jnp.float32),
        jax.ShapeDtypeStruct((NPAD, D), jnp.float32),
    ],
)


@jax.jit
def kernel(x, edge_index, W, b):
    src_p = jnp.concatenate([edge_index[0], jnp.zeros((PAD_E,), jnp.int32)])
    dst_p = jnp.concatenate([edge_index[1], jnp.full((PAD_E,), TRASH, jnp.int32)])
    ei_p = jnp.stack(
        [src_p.reshape(NW * CHUNKS, CH), dst_p.reshape(NW * CHUNKS, CH)], axis=1
    ).reshape(NW * CHUNKS * 2, CH)
    x_p = jnp.pad(x, ((0, NPAD - N), (0, 0)))
    wt = jnp.zeros((D, D), jnp.float32).at[:, :DOUT].set(W.T)
    bp = jnp.zeros((1, D), jnp.float32).at[0, :DOUT].set(b)
    zsrc = jnp.zeros((RPT, D), jnp.float32)

    degp = _deg_kernel(ei_p)
    g0, dinv, dginv = _prep(x_p, degp[:NPAD, None], degp[NPAD:, None])
    s1 = _scatter_kernel(g0, ei_p, zsrc)
    g1 = _combine(s1[:NPAD], s1[NPAD:], dginv)
    s2 = _scatter_kernel(g1, ei_p, zsrc)
    outp, embp = _final(s2[:NPAD], s2[NPAD:], dinv, wt, bp)
    return outp[:N, :DOUT], embp[:N, :DOUT]


# speedup vs baseline: 3.1658x; 3.1658x over previous
import functools

import jax
import jax.numpy as jnp
from jax import lax
from jax.experimental import pallas as pl
from jax.experimental.pallas import tpu as pltpu, tpu_sc as plsc

N = 10000
D = 128
DOUT = 40
E = 320000

NC = 2
NS = 16
NW = NC * NS
CH = 128
NPAD = 10240
RPT = NPAD // NS
PAIRS = 40
CHUNKS = 2 * PAIRS
TOT_E = NW * CH * CHUNKS
PAD_E = TOT_E - E
BR = 1024

_mesh = plsc.VectorSubcoreMesh(
    core_axis_name="c", subcore_axis_name="s", num_cores=NC, num_subcores=NS)


@functools.partial(
    pl.kernel,
    out_type=jax.ShapeDtypeStruct((NC * NPAD,), jnp.float32),
    mesh=_mesh,
    scratch_types=[
        pltpu.VMEM((2, CH), jnp.int32),
        pltpu.VMEM((2, CH), jnp.int32),
        pltpu.VMEM((CH,), jnp.float32),
        pltpu.VMEM((RPT,), jnp.float32),
        pltpu.VMEM_SHARED((NPAD,), jnp.float32),
        pltpu.SemaphoreType.DMA,
        pltpu.SemaphoreType.DMA,
        pltpu.SemaphoreType.DMA,
        pltpu.SemaphoreType.DMA,
    ],
)
def _deg_kernel(ei_hbm, out_hbm, iba, ibb, ones_v, zero_v, deg_s,
                isa, isb, ssa, ssb):
    cid = lax.axis_index("c")
    sid = lax.axis_index("s")
    wid = cid * NS + sid
    rbase = sid * RPT
    for i in range(CH // 16):
        ones_v[pl.ds(i * 16, 16)] = jnp.full((16,), 1.0, jnp.float32)
    for i in range(RPT // 16):
        zero_v[pl.ds(i * 16, 16)] = jnp.zeros((16,), jnp.float32)
    pltpu.sync_copy(zero_v, deg_s.at[pl.ds(rbase, RPT)])
    plsc.subcore_barrier()

    def body(i, carry):
        a = (wid * CHUNKS + 2 * i) * 2
        @pl.when(i > 0)
        def _():
            pltpu.make_async_copy(ones_v, deg_s.at[iba.at[1]], ssa).wait()
        pltpu.async_copy(ei_hbm.at[pl.ds(a, 2)], iba, isa)
        pltpu.make_async_copy(ei_hbm.at[pl.ds(a, 2)], iba, isa).wait()
        pltpu.async_copy(ones_v, deg_s.at[iba.at[1]], ssa, add=True)
        @pl.when(i > 0)
        def _():
            pltpu.make_async_copy(ones_v, deg_s.at[ibb.at[1]], ssb).wait()
        pltpu.async_copy(ei_hbm.at[pl.ds(a + 2, 2)], ibb, isb)
        pltpu.make_async_copy(ei_hbm.at[pl.ds(a + 2, 2)], ibb, isb).wait()
        pltpu.async_copy(ones_v, deg_s.at[ibb.at[1]], ssb, add=True)
        return carry

    lax.fori_loop(0, PAIRS, body, 0)
    pltpu.make_async_copy(ones_v, deg_s.at[iba.at[1]], ssa).wait()
    pltpu.make_async_copy(ones_v, deg_s.at[ibb.at[1]], ssb).wait()
    plsc.subcore_barrier()
    pltpu.sync_copy(deg_s.at[pl.ds(rbase, RPT)],
                    out_hbm.at[pl.ds(cid * NPAD + rbase, RPT)])


@functools.partial(
    pl.kernel,
    out_type=jax.ShapeDtypeStruct((NC * NPAD, D), jnp.float32),
    mesh=_mesh,
    scratch_types=[
        pltpu.VMEM((2, CH), jnp.int32),
        pltpu.VMEM((2, CH), jnp.int32),
        pltpu.VMEM((CH, D), jnp.float32),
        pltpu.VMEM((CH, D), jnp.float32),
        pltpu.VMEM_SHARED((NPAD, D), jnp.float32),
        pltpu.SemaphoreType.DMA,
        pltpu.SemaphoreType.DMA,
        pltpu.SemaphoreType.DMA,
        pltpu.SemaphoreType.DMA,
        pltpu.SemaphoreType.DMA,
        pltpu.SemaphoreType.DMA,
    ],
)
def _scatter_kernel(g_hbm, ei_hbm, zsrc_hbm, out_hbm,
                    iba, ibb, rowsa, rowsb, acc_s,
                    isa, isb, gsa, gsb, ssa, ssb):
    cid = lax.axis_index("c")
    sid = lax.axis_index("s")
    wid = cid * NS + sid
    rbase = sid * RPT

    @pl.when(cid == 0)
    def _():
        pltpu.sync_copy(g_hbm.at[pl.ds(rbase, RPT)], acc_s.at[pl.ds(rbase, RPT)])

    @pl.when(cid != 0)
    def _():
        pltpu.sync_copy(zsrc_hbm, acc_s.at[pl.ds(rbase, RPT)])

    plsc.subcore_barrier()

    def body(i, carry):
        a = (wid * CHUNKS + 2 * i) * 2
        @pl.when(i > 0)
        def _():
            pltpu.make_async_copy(rowsa, acc_s.at[iba.at[1]], ssa).wait()
        pltpu.async_copy(ei_hbm.at[pl.ds(a, 2)], iba, isa)
        pltpu.make_async_copy(ei_hbm.at[pl.ds(a, 2)], iba, isa).wait()
        pltpu.async_copy(g_hbm.at[iba.at[0]], rowsa, gsa)
        @pl.when(i > 0)
        def _():
            pltpu.make_async_copy(rowsb, acc_s.at[ibb.at[1]], ssb).wait()
        pltpu.async_copy(ei_hbm.at[pl.ds(a + 2, 2)], ibb, isb)
        pltpu.make_async_copy(ei_hbm.at[pl.ds(a + 2, 2)], ibb, isb).wait()
        pltpu.async_copy(g_hbm.at[ibb.at[0]], rowsb, gsb)
        pltpu.make_async_copy(g_hbm.at[iba.at[0]], rowsa, gsa).wait()
        pltpu.async_copy(rowsa, acc_s.at[iba.at[1]], ssa, add=True)
        pltpu.make_async_copy(g_hbm.at[ibb.at[0]], rowsb, gsb).wait()
        pltpu.async_copy(rowsb, acc_s.at[ibb.at[1]], ssb, add=True)
        return carry

    lax.fori_loop(0, PAIRS, body, 0)
    pltpu.make_async_copy(rowsa, acc_s.at[iba.at[1]], ssa).wait()
    pltpu.make_async_copy(rowsb, acc_s.at[ibb.at[1]], ssb).wait()
    plsc.subcore_barrier()
    pltpu.sync_copy(acc_s.at[pl.ds(rbase, RPT)],
                    out_hbm.at[pl.ds(cid * NPAD + rbase, RPT)])


def _prep_body(x_ref, d0_ref, d1_ref, g_ref, dinv_ref, dginv_ref):
    deg = d0_ref[...] + d1_ref[...] + 1.0
    dv = lax.rsqrt(deg)
    g_ref[...] = x_ref[...] * dv
    dinv_ref[...] = dv
    dginv_ref[...] = dv * dv


_prep = pl.pallas_call(
    _prep_body,
    grid=(NPAD // BR,),
    in_specs=[
        pl.BlockSpec((BR, D), lambda i: (i, 0)),
        pl.BlockSpec((BR, 1), lambda i: (i, 0)),
        pl.BlockSpec((BR, 1), lambda i: (i, 0)),
    ],
    out_specs=[
        pl.BlockSpec((BR, D), lambda i: (i, 0)),
        pl.BlockSpec((BR, 1), lambda i: (i, 0)),
        pl.BlockSpec((BR, 1), lambda i: (i, 0)),
    ],
    out_shape=[
        jax.ShapeDtypeStruct((NPAD, D), jnp.float32),
        jax.ShapeDtypeStruct((NPAD, 1), jnp.float32),
        jax.ShapeDtypeStruct((NPAD, 1), jnp.float32),
    ],
)


def _combine_body(s0_ref, s1_ref, dginv_ref, g_ref):
    g_ref[...] = (s0_ref[...] + s1_ref[...]) * dginv_ref[...]


_combine = pl.pallas_call(
    _combine_body,
    grid=(NPAD // BR,),
    in_specs=[
        pl.BlockSpec((BR, D), lambda i: (i, 0)),
        pl.BlockSpec((BR, D), lambda i: (i + NPAD // BR, 0)),
        pl.BlockSpec((BR, 1), lambda i: (i, 0)),
    ],
    out_specs=pl.BlockSpec((BR, D), lambda i: (i, 0)),
    out_shape=jax.ShapeDtypeStruct((NPAD, D), jnp.float32),
)


def _final_body(s0_ref, s1_ref, dinv_ref, wt_ref, b_ref, out_ref, emb_ref):
    h2 = (s0_ref[...] + s1_ref[...]) * dinv_ref[...]
    emb = jnp.dot(h2, wt_ref[...], preferred_element_type=jnp.float32) + b_ref[...]
    col = lax.broadcasted_iota(jnp.int32, emb.shape, 1)
    logits = jnp.where(col < DOUT, emb, -1e30)
    m = jnp.max(logits, axis=1, keepdims=True)
    lse = jnp.log(jnp.sum(jnp.exp(logits - m), axis=1, keepdims=True)) + m
    emb_ref[...] = emb
    out_ref[...] = emb - lse


_final = pl.pallas_call(
    _final_body,
    grid=(NPAD // BR,),
    in_specs=[
        pl.BlockSpec((BR, D), lambda i: (i, 0)),
        pl.BlockSpec((BR, D), lambda i: (i + NPAD // BR, 0)),
        pl.BlockSpec((BR, 1), lambda i: (i, 0)),
        pl.BlockSpec((D, D), lambda i: (0, 0)),
        pl.BlockSpec((1, D), lambda i: (0, 0)),
    ],
    out_specs=[
        pl.BlockSpec((BR, D), lambda i: (i, 0)),
        pl.BlockSpec((BR, D), lambda i: (i, 0)),
    ],
    out_shape=[
        jax.ShapeDtypeStruct((NPAD, D), jnp.float32),
        jax.ShapeDtypeStruct((NPAD, D), jnp.float32),
    ],
)


_PAD_SRC = 10000 + (jnp.arange(PAD_E, dtype=jnp.int32) % (NPAD - N))
_PAD_DST = 10000 + ((jnp.arange(PAD_E, dtype=jnp.int32) * 7 + 3) % (NPAD - N))


@jax.jit
def kernel(x, edge_index, W, b):
    src_p = jnp.concatenate([edge_index[0], _PAD_SRC])
    dst_p = jnp.concatenate([edge_index[1], _PAD_DST])
    ei_p = jnp.stack(
        [src_p.reshape(NW * CHUNKS, CH), dst_p.reshape(NW * CHUNKS, CH)], axis=1
    ).reshape(NW * CHUNKS * 2, CH)
    x_p = jnp.pad(x, ((0, NPAD - N), (0, 0)))
    wt = jnp.zeros((D, D), jnp.float32).at[:, :DOUT].set(W.T)
    bp = jnp.zeros((1, D), jnp.float32).at[0, :DOUT].set(b)
    zsrc = jnp.zeros((RPT, D), jnp.float32)

    degp = _deg_kernel(ei_p)
    g0, dinv, dginv = _prep(x_p, degp[:NPAD, None], degp[NPAD:, None])
    s1 = _scatter_kernel(g0, ei_p, zsrc)
    g1 = _combine(s1, s1, dginv)
    s2 = _scatter_kernel(g1, ei_p, zsrc)
    outp, embp = _final(s2, s2, dinv, wt, bp)
    return outp[:N, :DOUT], embp[:N, :DOUT]


# speedup vs baseline: 3.2770x; 1.0351x over previous
import functools

import jax
import jax.numpy as jnp
import numpy as np
from jax import lax
from jax.experimental import pallas as pl
from jax.experimental.pallas import tpu as pltpu, tpu_sc as plsc

N = 10000
D = 128
DOUT = 40
E = 320000

NC = 2
NS = 16
NW = NC * NS
CH = 128
NPAD = 10240
RPT = NPAD // NS
PAIRS = 40
CHUNKS = 2 * PAIRS
TOT_E = NW * CH * CHUNKS
PAD_E = TOT_E - E
BR = 1024

_mesh = plsc.VectorSubcoreMesh(
    core_axis_name="c", subcore_axis_name="s", num_cores=NC, num_subcores=NS)


@functools.partial(
    pl.kernel,
    out_type=jax.ShapeDtypeStruct((NC * NPAD,), jnp.float32),
    mesh=_mesh,
    scratch_types=[
        pltpu.VMEM((CHUNKS, CH), jnp.int32),
        pltpu.VMEM((CH,), jnp.int32),
        pltpu.VMEM((CH,), jnp.int32),
        pltpu.VMEM((CH,), jnp.float32),
        pltpu.VMEM((RPT,), jnp.float32),
        pltpu.VMEM_SHARED((NPAD,), jnp.float32),
        pltpu.SemaphoreType.DMA,
        pltpu.SemaphoreType.DMA,
    ],
)
def _deg_kernel(ei_hbm, out_hbm, iball, ixa, ixb, ones_v, zero_v, deg_s,
                ssa, ssb):
    cid = lax.axis_index("c")
    sid = lax.axis_index("s")
    wid = cid * NS + sid
    rbase = sid * RPT
    for i in range(CH // 16):
        ones_v[pl.ds(i * 16, 16)] = jnp.full((16,), 1.0, jnp.float32)
    for i in range(RPT // 16):
        zero_v[pl.ds(i * 16, 16)] = jnp.zeros((16,), jnp.float32)
    pltpu.sync_copy(ei_hbm.at[pl.ds(wid * CHUNKS, CHUNKS)], iball)
    pltpu.sync_copy(zero_v, deg_s.at[pl.ds(rbase, RPT)])
    plsc.subcore_barrier()

    def body(i, carry):
        @pl.when(i > 0)
        def _():
            pltpu.make_async_copy(ones_v, deg_s.at[ixa], ssa).wait()
        for j in range(CH // 16):
            ixa[pl.ds(j * 16, 16)] = (
                iball[2 * i, pl.ds(j * 16, 16)] >> 16)
        pltpu.async_copy(ones_v, deg_s.at[ixa], ssa, add=True)
        @pl.when(i > 0)
        def _():
            pltpu.make_async_copy(ones_v, deg_s.at[ixb], ssb).wait()
        for j in range(CH // 16):
            ixb[pl.ds(j * 16, 16)] = (
                iball[2 * i + 1, pl.ds(j * 16, 16)] >> 16)
        pltpu.async_copy(ones_v, deg_s.at[ixb], ssb, add=True)
        return carry

    lax.fori_loop(0, PAIRS, body, 0)
    pltpu.make_async_copy(ones_v, deg_s.at[ixa], ssa).wait()
    pltpu.make_async_copy(ones_v, deg_s.at[ixb], ssb).wait()
    plsc.subcore_barrier()
    pltpu.sync_copy(deg_s.at[pl.ds(rbase, RPT)],
                    out_hbm.at[pl.ds(cid * NPAD + rbase, RPT)])


@functools.partial(
    pl.kernel,
    out_type=jax.ShapeDtypeStruct((NC * NPAD, D), jnp.float32),
    mesh=_mesh,
    scratch_types=[
        pltpu.VMEM((CHUNKS, CH), jnp.int32),
        pltpu.VMEM((2, CH), jnp.int32),
        pltpu.VMEM((2, CH), jnp.int32),
        pltpu.VMEM((CH, D), jnp.float32),
        pltpu.VMEM((CH, D), jnp.float32),
        pltpu.VMEM_SHARED((NPAD, D), jnp.float32),
        pltpu.SemaphoreType.DMA,
        pltpu.SemaphoreType.DMA,
        pltpu.SemaphoreType.DMA,
        pltpu.SemaphoreType.DMA,
    ],
)
def _scatter_kernel(g_hbm, ei_hbm, zsrc_hbm, out_hbm,
                    iball, ixa, ixb, rowsa, rowsb, acc_s,
                    gsa, gsb, ssa, ssb):
    cid = lax.axis_index("c")
    sid = lax.axis_index("s")
    wid = cid * NS + sid
    rbase = sid * RPT

    pltpu.sync_copy(ei_hbm.at[pl.ds(wid * CHUNKS, CHUNKS)], iball)

    @pl.when(cid == 0)
    def _():
        pltpu.sync_copy(g_hbm.at[pl.ds(rbase, RPT)], acc_s.at[pl.ds(rbase, RPT)])

    @pl.when(cid != 0)
    def _():
        pltpu.sync_copy(zsrc_hbm, acc_s.at[pl.ds(rbase, RPT)])

    plsc.subcore_barrier()

    def body(i, carry):
        @pl.when(i > 0)
        def _():
            pltpu.make_async_copy(rowsa, acc_s.at[ixa.at[1]], ssa).wait()
        for j in range(CH // 16):
            v = iball[2 * i, pl.ds(j * 16, 16)]
            ixa[0, pl.ds(j * 16, 16)] = v & 0xFFFF
            ixa[1, pl.ds(j * 16, 16)] = v >> 16
        pltpu.async_copy(g_hbm.at[ixa.at[0]], rowsa, gsa)
        @pl.when(i > 0)
        def _():
            pltpu.make_async_copy(rowsb, acc_s.at[ixb.at[1]], ssb).wait()
        for j in range(CH // 16):
            v = iball[2 * i + 1, pl.ds(j * 16, 16)]
            ixb[0, pl.ds(j * 16, 16)] = v & 0xFFFF
            ixb[1, pl.ds(j * 16, 16)] = v >> 16
        pltpu.async_copy(g_hbm.at[ixb.at[0]], rowsb, gsb)
        pltpu.make_async_copy(g_hbm.at[ixa.at[0]], rowsa, gsa).wait()
        pltpu.async_copy(rowsa, acc_s.at[ixa.at[1]], ssa, add=True)
        pltpu.make_async_copy(g_hbm.at[ixb.at[0]], rowsb, gsb).wait()
        pltpu.async_copy(rowsb, acc_s.at[ixb.at[1]], ssb, add=True)
        return carry

    lax.fori_loop(0, PAIRS, body, 0)
    pltpu.make_async_copy(rowsa, acc_s.at[ixa.at[1]], ssa).wait()
    pltpu.make_async_copy(rowsb, acc_s.at[ixb.at[1]], ssb).wait()
    plsc.subcore_barrier()
    pltpu.sync_copy(acc_s.at[pl.ds(rbase, RPT)],
                    out_hbm.at[pl.ds(cid * NPAD + rbase, RPT)])


def _prep_body(x_ref, d0_ref, d1_ref, g_ref, dinv_ref, dginv_ref):
    deg = d0_ref[...] + d1_ref[...] + 1.0
    dv = lax.rsqrt(deg)
    g_ref[...] = x_ref[...] * dv
    dinv_ref[...] = dv
    dginv_ref[...] = dv * dv


_prep = pl.pallas_call(
    _prep_body,
    grid=(NPAD // BR,),
    in_specs=[
        pl.BlockSpec((BR, D), lambda i: (i, 0)),
        pl.BlockSpec((BR, 1), lambda i: (i, 0)),
        pl.BlockSpec((BR, 1), lambda i: (i, 0)),
    ],
    out_specs=[
        pl.BlockSpec((BR, D), lambda i: (i, 0)),
        pl.BlockSpec((BR, 1), lambda i: (i, 0)),
        pl.BlockSpec((BR, 1), lambda i: (i, 0)),
    ],
    out_shape=[
        jax.ShapeDtypeStruct((NPAD, D), jnp.float32),
        jax.ShapeDtypeStruct((NPAD, 1), jnp.float32),
        jax.ShapeDtypeStruct((NPAD, 1), jnp.float32),
    ],
)


def _combine_body(s0_ref, s1_ref, dginv_ref, g_ref):
    g_ref[...] = (s0_ref[...] + s1_ref[...]) * dginv_ref[...]


_combine = pl.pallas_call(
    _combine_body,
    grid=(NPAD // BR,),
    in_specs=[
        pl.BlockSpec((BR, D), lambda i: (i, 0)),
        pl.BlockSpec((BR, D), lambda i: (i + NPAD // BR, 0)),
        pl.BlockSpec((BR, 1), lambda i: (i, 0)),
    ],
    out_specs=pl.BlockSpec((BR, D), lambda i: (i, 0)),
    out_shape=jax.ShapeDtypeStruct((NPAD, D), jnp.float32),
)


def _final_body(s0_ref, s1_ref, dinv_ref, wt_ref, b_ref, out_ref, emb_ref):
    h2 = (s0_ref[...] + s1_ref[...]) * dinv_ref[...]
    emb = jnp.dot(h2, wt_ref[...], preferred_element_type=jnp.float32) + b_ref[...]
    col = lax.broadcasted_iota(jnp.int32, emb.shape, 1)
    logits = jnp.where(col < DOUT, emb, -1e30)
    m = jnp.max(logits, axis=1, keepdims=True)
    lse = jnp.log(jnp.sum(jnp.exp(logits - m), axis=1, keepdims=True)) + m
    emb_ref[...] = emb
    out_ref[...] = emb - lse


_final = pl.pallas_call(
    _final_body,
    grid=(NPAD // BR,),
    in_specs=[
        pl.BlockSpec((BR, D), lambda i: (i, 0)),
        pl.BlockSpec((BR, D), lambda i: (i + NPAD // BR, 0)),
        pl.BlockSpec((BR, 1), lambda i: (i, 0)),
        pl.BlockSpec((D, D), lambda i: (0, 0)),
        pl.BlockSpec((1, D), lambda i: (0, 0)),
    ],
    out_specs=[
        pl.BlockSpec((BR, D), lambda i: (i, 0)),
        pl.BlockSpec((BR, D), lambda i: (i, 0)),
    ],
    out_shape=[
        jax.ShapeDtypeStruct((NPAD, D), jnp.float32),
        jax.ShapeDtypeStruct((NPAD, D), jnp.float32),
    ],
)


_PAD_SRC = np.int32(10000) + (np.arange(PAD_E, dtype=np.int32) % (NPAD - N))
_PAD_DST = np.int32(10000) + ((np.arange(PAD_E, dtype=np.int32) * 7 + 3) % (NPAD - N))


@jax.jit
def kernel(x, edge_index, W, b):
    src_p = jnp.concatenate([edge_index[0], _PAD_SRC])
    dst_p = jnp.concatenate([edge_index[1], _PAD_DST])
    ei_p = (src_p + dst_p * 65536).reshape(NW * CHUNKS, CH)
    x_p = jnp.pad(x, ((0, NPAD - N), (0, 0)))
    wt = jnp.zeros((D, D), jnp.float32).at[:, :DOUT].set(W.T)
    bp = jnp.zeros((1, D), jnp.float32).at[0, :DOUT].set(b)
    zsrc = jnp.zeros((RPT, D), jnp.float32)

    degp = _deg_kernel(ei_p)
    g0, dinv, dginv = _prep(x_p, degp[:NPAD, None], degp[NPAD:, None])
    s1 = _scatter_kernel(g0, ei_p, zsrc)
    g1 = _combine(s1, s1, dginv)
    s2 = _scatter_kernel(g1, ei_p, zsrc)
    outp, embp = _final(s2, s2, dinv, wt, bp)
    return outp[:N, :DOUT], embp[:N, :DOUT]


# speedup vs baseline: 3.9027x; 1.1910x over previous
import functools

import jax
import jax.numpy as jnp
import numpy as np
from jax import lax
from jax.experimental import pallas as pl
from jax.experimental.pallas import tpu as pltpu, tpu_sc as plsc

N = 10000
D = 128
DOUT = 40
E = 320000

NC = 2
NS = 16
NW = NC * NS
CH = 128
NPAD = 10240
RPT = NPAD // NS
PAIRS = 40
CHUNKS = 2 * PAIRS
TOT_E = NW * CH * CHUNKS
PAD_E = TOT_E - E
BR = 1024

_mesh = plsc.VectorSubcoreMesh(
    core_axis_name="c", subcore_axis_name="s", num_cores=NC, num_subcores=NS)


@functools.partial(
    pl.kernel,
    out_type=jax.ShapeDtypeStruct((NC * NPAD,), jnp.float32),
    mesh=_mesh,
    scratch_types=[
        pltpu.VMEM((CHUNKS, CH), jnp.int32),
        pltpu.VMEM((CH,), jnp.int32),
        pltpu.VMEM((CH,), jnp.int32),
        pltpu.VMEM((CH,), jnp.float32),
        pltpu.VMEM((RPT,), jnp.float32),
        pltpu.VMEM_SHARED((NPAD,), jnp.float32),
        pltpu.SemaphoreType.DMA,
        pltpu.SemaphoreType.DMA,
    ],
)
def _deg_kernel(ei_hbm, out_hbm, iball, ixa, ixb, ones_v, zero_v, deg_s,
                ssa, ssb):
    cid = lax.axis_index("c")
    sid = lax.axis_index("s")
    wid = cid * NS + sid
    rbase = sid * RPT
    for i in range(CH // 16):
        ones_v[pl.ds(i * 16, 16)] = jnp.full((16,), 1.0, jnp.float32)
    for i in range(RPT // 16):
        zero_v[pl.ds(i * 16, 16)] = jnp.zeros((16,), jnp.float32)
    pltpu.sync_copy(ei_hbm.at[pl.ds(wid * CHUNKS, CHUNKS)], iball)
    pltpu.sync_copy(zero_v, deg_s.at[pl.ds(rbase, RPT)])
    plsc.subcore_barrier()

    def body(i, carry):
        @pl.when(i > 0)
        def _():
            pltpu.make_async_copy(ones_v, deg_s.at[ixa], ssa).wait()
        for j in range(CH // 16):
            ixa[pl.ds(j * 16, 16)] = (
                iball[2 * i, pl.ds(j * 16, 16)] >> 16)
        pltpu.async_copy(ones_v, deg_s.at[ixa], ssa, add=True)
        @pl.when(i > 0)
        def _():
            pltpu.make_async_copy(ones_v, deg_s.at[ixb], ssb).wait()
        for j in range(CH // 16):
            ixb[pl.ds(j * 16, 16)] = (
                iball[2 * i + 1, pl.ds(j * 16, 16)] >> 16)
        pltpu.async_copy(ones_v, deg_s.at[ixb], ssb, add=True)
        return carry

    lax.fori_loop(0, PAIRS, body, 0)
    pltpu.make_async_copy(ones_v, deg_s.at[ixa], ssa).wait()
    pltpu.make_async_copy(ones_v, deg_s.at[ixb], ssb).wait()
    plsc.subcore_barrier()
    pltpu.sync_copy(deg_s.at[pl.ds(rbase, RPT)],
                    out_hbm.at[pl.ds(cid * NPAD + rbase, RPT)])


SCH = 64
SQUADS = CHUNKS // 2
STAGES = 4


@functools.partial(
    pl.kernel,
    out_type=jax.ShapeDtypeStruct((NC * NPAD, D), jnp.float32),
    mesh=_mesh,
    scratch_types=[
        pltpu.VMEM((CHUNKS, CH), jnp.int32),
        pltpu.VMEM((2, SCH), jnp.int32),
        pltpu.VMEM((2, SCH), jnp.int32),
        pltpu.VMEM((2, SCH), jnp.int32),
        pltpu.VMEM((2, SCH), jnp.int32),
        pltpu.VMEM((SCH, D), jnp.float32),
        pltpu.VMEM((SCH, D), jnp.float32),
        pltpu.VMEM((SCH, D), jnp.float32),
        pltpu.VMEM((SCH, D), jnp.float32),
        pltpu.VMEM_SHARED((NPAD, D), jnp.float32),
        pltpu.SemaphoreType.DMA,
        pltpu.SemaphoreType.DMA,
        pltpu.SemaphoreType.DMA,
        pltpu.SemaphoreType.DMA,
        pltpu.SemaphoreType.DMA,
        pltpu.SemaphoreType.DMA,
        pltpu.SemaphoreType.DMA,
        pltpu.SemaphoreType.DMA,
    ],
)
def _scatter_kernel(g_hbm, ei_hbm, zsrc_hbm, out_hbm,
                    iball, ix0, ix1, ix2, ix3, rw0, rw1, rw2, rw3, acc_s,
                    gs0, gs1, gs2, gs3, ss0, ss1, ss2, ss3):
    cid = lax.axis_index("c")
    sid = lax.axis_index("s")
    wid = cid * NS + sid
    rbase = sid * RPT
    ix = (ix0, ix1, ix2, ix3)
    rw = (rw0, rw1, rw2, rw3)
    gs = (gs0, gs1, gs2, gs3)
    ss = (ss0, ss1, ss2, ss3)

    pltpu.sync_copy(ei_hbm.at[pl.ds(wid * CHUNKS, CHUNKS)], iball)

    @pl.when(cid == 0)
    def _():
        pltpu.sync_copy(g_hbm.at[pl.ds(rbase, RPT)], acc_s.at[pl.ds(rbase, RPT)])

    @pl.when(cid != 0)
    def _():
        pltpu.sync_copy(zsrc_hbm, acc_s.at[pl.ds(rbase, RPT)])

    plsc.subcore_barrier()

    def body(i, carry):
        for k in range(STAGES):
            r = 2 * i + k // 2
            off = (k % 2) * SCH
            @pl.when(i > 0)
            def _():
                pltpu.make_async_copy(rw[k], acc_s.at[ix[k].at[1]], ss[k]).wait()
            for j in range(SCH // 16):
                v = iball[r, pl.ds(off + j * 16, 16)]
                ix[k][0, pl.ds(j * 16, 16)] = v & 0xFFFF
                ix[k][1, pl.ds(j * 16, 16)] = v >> 16
            pltpu.async_copy(g_hbm.at[ix[k].at[0]], rw[k], gs[k])
        for k in range(STAGES):
            pltpu.make_async_copy(g_hbm.at[ix[k].at[0]], rw[k], gs[k]).wait()
            pltpu.async_copy(rw[k], acc_s.at[ix[k].at[1]], ss[k], add=True)
        return carry

    lax.fori_loop(0, SQUADS, body, 0)
    for k in range(STAGES):
        pltpu.make_async_copy(rw[k], acc_s.at[ix[k].at[1]], ss[k]).wait()
    plsc.subcore_barrier()
    pltpu.sync_copy(acc_s.at[pl.ds(rbase, RPT)],
                    out_hbm.at[pl.ds(cid * NPAD + rbase, RPT)])


def _prep_body(x_ref, d0_ref, d1_ref, g_ref, dinv_ref, dginv_ref):
    deg = d0_ref[...] + d1_ref[...] + 1.0
    dv = lax.rsqrt(deg)
    g_ref[...] = x_ref[...] * dv
    dinv_ref[...] = dv
    dginv_ref[...] = dv * dv


_prep = pl.pallas_call(
    _prep_body,
    grid=(NPAD // BR,),
    in_specs=[
        pl.BlockSpec((BR, D), lambda i: (i, 0)),
        pl.BlockSpec((BR, 1), lambda i: (i, 0)),
        pl.BlockSpec((BR, 1), lambda i: (i, 0)),
    ],
    out_specs=[
        pl.BlockSpec((BR, D), lambda i: (i, 0)),
        pl.BlockSpec((BR, 1), lambda i: (i, 0)),
        pl.BlockSpec((BR, 1), lambda i: (i, 0)),
    ],
    out_shape=[
        jax.ShapeDtypeStruct((NPAD, D), jnp.float32),
        jax.ShapeDtypeStruct((NPAD, 1), jnp.float32),
        jax.ShapeDtypeStruct((NPAD, 1), jnp.float32),
    ],
)


def _combine_body(s0_ref, s1_ref, dginv_ref, g_ref):
    g_ref[...] = (s0_ref[...] + s1_ref[...]) * dginv_ref[...]


_combine = pl.pallas_call(
    _combine_body,
    grid=(NPAD // BR,),
    in_specs=[
        pl.BlockSpec((BR, D), lambda i: (i, 0)),
        pl.BlockSpec((BR, D), lambda i: (i + NPAD // BR, 0)),
        pl.BlockSpec((BR, 1), lambda i: (i, 0)),
    ],
    out_specs=pl.BlockSpec((BR, D), lambda i: (i, 0)),
    out_shape=jax.ShapeDtypeStruct((NPAD, D), jnp.float32),
)


def _final_body(s0_ref, s1_ref, dinv_ref, wt_ref, b_ref, out_ref, emb_ref):
    h2 = (s0_ref[...] + s1_ref[...]) * dinv_ref[...]
    emb = jnp.dot(h2, wt_ref[...], preferred_element_type=jnp.float32) + b_ref[...]
    col = lax.broadcasted_iota(jnp.int32, emb.shape, 1)
    logits = jnp.where(col < DOUT, emb, -1e30)
    m = jnp.max(logits, axis=1, keepdims=True)
    lse = jnp.log(jnp.sum(jnp.exp(logits - m), axis=1, keepdims=True)) + m
    emb_ref[...] = emb
    out_ref[...] = emb - lse


_final = pl.pallas_call(
    _final_body,
    grid=(NPAD // BR,),
    in_specs=[
        pl.BlockSpec((BR, D), lambda i: (i, 0)),
        pl.BlockSpec((BR, D), lambda i: (i + NPAD // BR, 0)),
        pl.BlockSpec((BR, 1), lambda i: (i, 0)),
        pl.BlockSpec((D, D), lambda i: (0, 0)),
        pl.BlockSpec((1, D), lambda i: (0, 0)),
    ],
    out_specs=[
        pl.BlockSpec((BR, D), lambda i: (i, 0)),
        pl.BlockSpec((BR, D), lambda i: (i, 0)),
    ],
    out_shape=[
        jax.ShapeDtypeStruct((NPAD, D), jnp.float32),
        jax.ShapeDtypeStruct((NPAD, D), jnp.float32),
    ],
)


_PAD_SRC = np.int32(10000) + (np.arange(PAD_E, dtype=np.int32) % (NPAD - N))
_PAD_DST = np.int32(10000) + ((np.arange(PAD_E, dtype=np.int32) * 7 + 3) % (NPAD - N))


@jax.jit
def kernel(x, edge_index, W, b):
    src_p = jnp.concatenate([edge_index[0], _PAD_SRC])
    dst_p = jnp.concatenate([edge_index[1], _PAD_DST])
    ei_p = (src_p + dst_p * 65536).reshape(NW * CHUNKS, CH)
    x_p = jnp.pad(x, ((0, NPAD - N), (0, 0)))
    wt = jnp.zeros((D, D), jnp.float32).at[:, :DOUT].set(W.T)
    bp = jnp.zeros((1, D), jnp.float32).at[0, :DOUT].set(b)
    zsrc = jnp.zeros((RPT, D), jnp.float32)

    degp = _deg_kernel(ei_p)
    g0, dinv, dginv = _prep(x_p, degp[:NPAD, None], degp[NPAD:, None])
    s1 = _scatter_kernel(g0, ei_p, zsrc)
    g1 = _combine(s1, s1, dginv)
    s2 = _scatter_kernel(g1, ei_p, zsrc)
    outp, embp = _final(s2, s2, dinv, wt, bp)
    return outp[:N, :DOUT], embp[:N, :DOUT]


# speedup vs baseline: 3.9640x; 1.0157x over previous
import functools

import jax
import jax.numpy as jnp
import numpy as np
from jax import lax
from jax.experimental import pallas as pl
from jax.experimental.pallas import tpu as pltpu, tpu_sc as plsc

N = 10000
D = 128
DOUT = 40
E = 320000

NC = 2
NS = 16
NW = NC * NS
CH = 128
NPAD = 10240
RPT = NPAD // NS
PAIRS = 40
CHUNKS = 2 * PAIRS
TOT_E = NW * CH * CHUNKS
PAD_E = TOT_E - E
BR = 1024

_mesh = plsc.VectorSubcoreMesh(
    core_axis_name="c", subcore_axis_name="s", num_cores=NC, num_subcores=NS)


@functools.partial(
    pl.kernel,
    out_type=jax.ShapeDtypeStruct((NC * NPAD,), jnp.float32),
    mesh=_mesh,
    scratch_types=[
        pltpu.VMEM((CHUNKS, CH), jnp.int32),
        pltpu.VMEM((CH,), jnp.int32),
        pltpu.VMEM((CH,), jnp.int32),
        pltpu.VMEM((CH,), jnp.float32),
        pltpu.VMEM((RPT,), jnp.float32),
        pltpu.VMEM_SHARED((NPAD,), jnp.float32),
        pltpu.SemaphoreType.DMA,
        pltpu.SemaphoreType.DMA,
    ],
)
def _deg_kernel(ei_hbm, out_hbm, iball, ixa, ixb, ones_v, zero_v, deg_s,
                ssa, ssb):
    cid = lax.axis_index("c")
    sid = lax.axis_index("s")
    wid = cid * NS + sid
    rbase = sid * RPT
    for i in range(CH // 16):
        ones_v[pl.ds(i * 16, 16)] = jnp.full((16,), 1.0, jnp.float32)
    for i in range(RPT // 16):
        zero_v[pl.ds(i * 16, 16)] = jnp.zeros((16,), jnp.float32)
    pltpu.sync_copy(ei_hbm.at[pl.ds(wid * CHUNKS, CHUNKS)], iball)
    pltpu.sync_copy(zero_v, deg_s.at[pl.ds(rbase, RPT)])
    plsc.subcore_barrier()

    def body(i, carry):
        @pl.when(i > 0)
        def _():
            pltpu.make_async_copy(ones_v, deg_s.at[ixa], ssa).wait()
        for j in range(CH // 16):
            ixa[pl.ds(j * 16, 16)] = (
                iball[2 * i, pl.ds(j * 16, 16)] >> 16)
        pltpu.async_copy(ones_v, deg_s.at[ixa], ssa, add=True)
        @pl.when(i > 0)
        def _():
            pltpu.make_async_copy(ones_v, deg_s.at[ixb], ssb).wait()
        for j in range(CH // 16):
            ixb[pl.ds(j * 16, 16)] = (
                iball[2 * i + 1, pl.ds(j * 16, 16)] >> 16)
        pltpu.async_copy(ones_v, deg_s.at[ixb], ssb, add=True)
        return carry

    lax.fori_loop(0, PAIRS, body, 0)
    pltpu.make_async_copy(ones_v, deg_s.at[ixa], ssa).wait()
    pltpu.make_async_copy(ones_v, deg_s.at[ixb], ssb).wait()
    plsc.subcore_barrier()
    pltpu.sync_copy(deg_s.at[pl.ds(rbase, RPT)],
                    out_hbm.at[pl.ds(cid * NPAD + rbase, RPT)])


SCH = 32
SQUADS = CHUNKS // 2
STAGES = 8


@functools.partial(
    pl.kernel,
    out_type=jax.ShapeDtypeStruct((NC * NPAD, D), jnp.float32),
    mesh=_mesh,
    scratch_types=[
        pltpu.VMEM((CHUNKS, CH), jnp.int32),
        pltpu.VMEM((2, SCH), jnp.int32),
        pltpu.VMEM((2, SCH), jnp.int32),
        pltpu.VMEM((2, SCH), jnp.int32),
        pltpu.VMEM((2, SCH), jnp.int32),
        pltpu.VMEM((2, SCH), jnp.int32),
        pltpu.VMEM((2, SCH), jnp.int32),
        pltpu.VMEM((2, SCH), jnp.int32),
        pltpu.VMEM((2, SCH), jnp.int32),
        pltpu.VMEM((SCH, D), jnp.float32),
        pltpu.VMEM((SCH, D), jnp.float32),
        pltpu.VMEM((SCH, D), jnp.float32),
        pltpu.VMEM((SCH, D), jnp.float32),
        pltpu.VMEM((SCH, D), jnp.float32),
        pltpu.VMEM((SCH, D), jnp.float32),
        pltpu.VMEM((SCH, D), jnp.float32),
        pltpu.VMEM((SCH, D), jnp.float32),
        pltpu.VMEM_SHARED((NPAD, D), jnp.float32),
        pltpu.SemaphoreType.DMA,
        pltpu.SemaphoreType.DMA,
        pltpu.SemaphoreType.DMA,
        pltpu.SemaphoreType.DMA,
        pltpu.SemaphoreType.DMA,
        pltpu.SemaphoreType.DMA,
        pltpu.SemaphoreType.DMA,
        pltpu.SemaphoreType.DMA,
        pltpu.SemaphoreType.DMA,
        pltpu.SemaphoreType.DMA,
        pltpu.SemaphoreType.DMA,
        pltpu.SemaphoreType.DMA,
        pltpu.SemaphoreType.DMA,
        pltpu.SemaphoreType.DMA,
        pltpu.SemaphoreType.DMA,
        pltpu.SemaphoreType.DMA,
    ],
)
def _scatter_kernel(g_hbm, ei_hbm, zsrc_hbm, out_hbm,
                    iball, ix0, ix1, ix2, ix3, ix4, ix5, ix6, ix7,
                    rw0, rw1, rw2, rw3, rw4, rw5, rw6, rw7, acc_s,
                    gs0, gs1, gs2, gs3, gs4, gs5, gs6, gs7,
                    ss0, ss1, ss2, ss3, ss4, ss5, ss6, ss7):
    cid = lax.axis_index("c")
    sid = lax.axis_index("s")
    wid = cid * NS + sid
    rbase = sid * RPT
    ix = (ix0, ix1, ix2, ix3, ix4, ix5, ix6, ix7)
    rw = (rw0, rw1, rw2, rw3, rw4, rw5, rw6, rw7)
    gs = (gs0, gs1, gs2, gs3, gs4, gs5, gs6, gs7)
    ss = (ss0, ss1, ss2, ss3, ss4, ss5, ss6, ss7)

    pltpu.sync_copy(ei_hbm.at[pl.ds(wid * CHUNKS, CHUNKS)], iball)

    @pl.when(cid == 0)
    def _():
        pltpu.sync_copy(g_hbm.at[pl.ds(rbase, RPT)], acc_s.at[pl.ds(rbase, RPT)])

    @pl.when(cid != 0)
    def _():
        pltpu.sync_copy(zsrc_hbm, acc_s.at[pl.ds(rbase, RPT)])

    plsc.subcore_barrier()

    def body(i, carry):
        for k in range(STAGES):
            r = 2 * i + k // 4
            off = (k % 4) * SCH
            @pl.when(i > 0)
            def _():
                pltpu.make_async_copy(rw[k], acc_s.at[ix[k].at[1]], ss[k]).wait()
            for j in range(SCH // 16):
                v = iball[r, pl.ds(off + j * 16, 16)]
                ix[k][0, pl.ds(j * 16, 16)] = v & 0xFFFF
                ix[k][1, pl.ds(j * 16, 16)] = v >> 16
            pltpu.async_copy(g_hbm.at[ix[k].at[0]], rw[k], gs[k])
        for k in range(STAGES):
            pltpu.make_async_copy(g_hbm.at[ix[k].at[0]], rw[k], gs[k]).wait()
            pltpu.async_copy(rw[k], acc_s.at[ix[k].at[1]], ss[k], add=True)
        return carry

    lax.fori_loop(0, SQUADS, body, 0)
    for k in range(STAGES):
        pltpu.make_async_copy(rw[k], acc_s.at[ix[k].at[1]], ss[k]).wait()
    plsc.subcore_barrier()
    pltpu.sync_copy(acc_s.at[pl.ds(rbase, RPT)],
                    out_hbm.at[pl.ds(cid * NPAD + rbase, RPT)])


def _prep_body(x_ref, d0_ref, d1_ref, g_ref, dinv_ref, dginv_ref):
    deg = d0_ref[...] + d1_ref[...] + 1.0
    dv = lax.rsqrt(deg)
    g_ref[...] = x_ref[...] * dv
    dinv_ref[...] = dv
    dginv_ref[...] = dv * dv


_prep = pl.pallas_call(
    _prep_body,
    grid=(NPAD // BR,),
    in_specs=[
        pl.BlockSpec((BR, D), lambda i: (i, 0)),
        pl.BlockSpec((BR, 1), lambda i: (i, 0)),
        pl.BlockSpec((BR, 1), lambda i: (i, 0)),
    ],
    out_specs=[
        pl.BlockSpec((BR, D), lambda i: (i, 0)),
        pl.BlockSpec((BR, 1), lambda i: (i, 0)),
        pl.BlockSpec((BR, 1), lambda i: (i, 0)),
    ],
    out_shape=[
        jax.ShapeDtypeStruct((NPAD, D), jnp.float32),
        jax.ShapeDtypeStruct((NPAD, 1), jnp.float32),
        jax.ShapeDtypeStruct((NPAD, 1), jnp.float32),
    ],
)


def _combine_body(s0_ref, s1_ref, dginv_ref, g_ref):
    g_ref[...] = (s0_ref[...] + s1_ref[...]) * dginv_ref[...]


_combine = pl.pallas_call(
    _combine_body,
    grid=(NPAD // BR,),
    in_specs=[
        pl.BlockSpec((BR, D), lambda i: (i, 0)),
        pl.BlockSpec((BR, D), lambda i: (i + NPAD // BR, 0)),
        pl.BlockSpec((BR, 1), lambda i: (i, 0)),
    ],
    out_specs=pl.BlockSpec((BR, D), lambda i: (i, 0)),
    out_shape=jax.ShapeDtypeStruct((NPAD, D), jnp.float32),
)


def _final_body(s0_ref, s1_ref, dinv_ref, wt_ref, b_ref, out_ref, emb_ref):
    h2 = (s0_ref[...] + s1_ref[...]) * dinv_ref[...]
    emb = jnp.dot(h2, wt_ref[...], preferred_element_type=jnp.float32) + b_ref[...]
    col = lax.broadcasted_iota(jnp.int32, emb.shape, 1)
    logits = jnp.where(col < DOUT, emb, -1e30)
    m = jnp.max(logits, axis=1, keepdims=True)
    lse = jnp.log(jnp.sum(jnp.exp(logits - m), axis=1, keepdims=True)) + m
    emb_ref[...] = emb
    out_ref[...] = emb - lse


_final = pl.pallas_call(
    _final_body,
    grid=(NPAD // BR,),
    in_specs=[
        pl.BlockSpec((BR, D), lambda i: (i, 0)),
        pl.BlockSpec((BR, D), lambda i: (i + NPAD // BR, 0)),
        pl.BlockSpec((BR, 1), lambda i: (i, 0)),
        pl.BlockSpec((D, D), lambda i: (0, 0)),
        pl.BlockSpec((1, D), lambda i: (0, 0)),
    ],
    out_specs=[
        pl.BlockSpec((BR, D), lambda i: (i, 0)),
        pl.BlockSpec((BR, D), lambda i: (i, 0)),
    ],
    out_shape=[
        jax.ShapeDtypeStruct((NPAD, D), jnp.float32),
        jax.ShapeDtypeStruct((NPAD, D), jnp.float32),
    ],
)


_PAD_SRC = np.int32(10000) + (np.arange(PAD_E, dtype=np.int32) % (NPAD - N))
_PAD_DST = np.int32(10000) + ((np.arange(PAD_E, dtype=np.int32) * 7 + 3) % (NPAD - N))


@jax.jit
def kernel(x, edge_index, W, b):
    src_p = jnp.concatenate([edge_index[0], _PAD_SRC])
    dst_p = jnp.concatenate([edge_index[1], _PAD_DST])
    ei_p = (src_p + dst_p * 65536).reshape(NW * CHUNKS, CH)
    x_p = jnp.pad(x, ((0, NPAD - N), (0, 0)))
    wt = jnp.zeros((D, D), jnp.float32).at[:, :DOUT].set(W.T)
    bp = jnp.zeros((1, D), jnp.float32).at[0, :DOUT].set(b)
    zsrc = jnp.zeros((RPT, D), jnp.float32)

    degp = _deg_kernel(ei_p)
    g0, dinv, dginv = _prep(x_p, degp[:NPAD, None], degp[NPAD:, None])
    s1 = _scatter_kernel(g0, ei_p, zsrc)
    g1 = _combine(s1, s1, dginv)
    s2 = _scatter_kernel(g1, ei_p, zsrc)
    outp, embp = _final(s2, s2, dinv, wt, bp)
    return outp[:N, :DOUT], embp[:N, :DOUT]
